# Initial kernel scaffold; baseline (speedup 1.0000x reference)
import jax
import jax.numpy as jnp
from jax.experimental import pallas as pl


def kernel(edge_index, imagefeatures, wordfeatures, descfeatures, genrefeatures, compfeatures, gat_params, mean_params, logstd_params, sem_params):
    raise NotImplementedError("write your pallas kernel here")


# speedup vs baseline: 16.4417x; 16.4417x over previous
import jax
import jax.numpy as jnp
from jax import lax
from jax.experimental import pallas as pl
from jax.experimental.pallas import tpu as pltpu
from jax.experimental.pallas import tpu_sc as plsc

N = 10000
E = 320000
D = 128
H1 = 128
H2 = 64

NC = 2
NS = 16
L = 16
NW = NC * NS

NPAD = 10240
RPT = NPAD // NS
EB = 128
NBATCH = 79
EW = EB * NBATCH
EPAD = EW * NW

ROW_BLK = 1024
GRID = NPAD // ROW_BLK

_MESH = plsc.VectorSubcoreMesh(
    core_axis_name="c", subcore_axis_name="s", num_cores=NC, num_subcores=NS)


def _edge_body(h_hbm, el_hbm, er_hbm, src_hbm, dst_hbm,
               acc_hbm, s_hbm,
               el_v, er_v, src_v, dst_v, ex_v, rows_v, s_loc, acc_sh, sem):
    cid = lax.axis_index("c")
    sid = lax.axis_index("s")
    wid = sid * NC + cid

    pltpu.sync_copy(el_hbm, el_v)
    pltpu.sync_copy(er_hbm, er_v)

    zeros16 = jnp.zeros((L,), jnp.float32)

    def _zrow(i, carry):
        for c in range(H1 // L):
            rows_v[i, pl.ds(c * L, L)] = zeros16
        return carry
    lax.fori_loop(0, EB, _zrow, 0)

    def _zs(i, carry):
        s_loc[pl.ds(i * L, L)] = zeros16
        return carry
    lax.fori_loop(0, NPAD // L, _zs, 0)

    r0 = sid * RPT
    for c in range(RPT // EB):
        pltpu.sync_copy(rows_v, acc_sh.at[pl.ds(r0 + c * EB, EB)])
    plsc.subcore_barrier()

    ebase = wid * EW

    def _batch(bi, carry):
        base = ebase + bi * EB
        pltpu.sync_copy(src_hbm.at[pl.ds(base, EB)], src_v)
        pltpu.sync_copy(dst_hbm.at[pl.ds(base, EB)], dst_v)
        pltpu.async_copy(h_hbm.at[src_v], rows_v, sem).wait()
        for g in range(EB // L):
            sl = pl.ds(g * L, L)
            sv = src_v[sl]
            dv = dst_v[sl]
            e = plsc.load_gather(el_v, [sv]) + plsc.load_gather(er_v, [dv])
            e = jnp.where(e > 0.0, e, 0.1 * e)
            ex = jnp.exp(e)
            ex_v[sl] = ex
            plsc.addupdate_scatter(s_loc, [dv], ex)

        def _scale(j, c2):
            exb = plsc.load_gather(ex_v, [jnp.full((L,), j, jnp.int32)])
            for c in range(H1 // L):
                slc = pl.ds(c * L, L)
                rows_v[j, slc] = rows_v[j, slc] * exb
            return c2
        lax.fori_loop(0, EB, _scale, 0)

        pltpu.sync_copy(rows_v, acc_sh.at[dst_v], add=True)
        return carry
    lax.fori_loop(0, NBATCH, _batch, 0)

    pltpu.sync_copy(s_loc, s_hbm.at[pl.ds(wid * NPAD, NPAD)])
    plsc.subcore_barrier()

    out0 = cid * NPAD + r0
    for c in range(RPT // EB):
        pltpu.sync_copy(acc_sh.at[pl.ds(r0 + c * EB, EB)],
                        acc_hbm.at[pl.ds(out0 + c * EB, EB)])


_edge_call = pl.kernel(
    _edge_body,
    out_type=(jax.ShapeDtypeStruct((NC * NPAD, H1), jnp.float32),
              jax.ShapeDtypeStruct((NW * NPAD,), jnp.float32)),
    mesh=_MESH,
    scratch_types=[
        pltpu.VMEM((NPAD,), jnp.float32),
        pltpu.VMEM((NPAD,), jnp.float32),
        pltpu.VMEM((EB,), jnp.int32),
        pltpu.VMEM((EB,), jnp.int32),
        pltpu.VMEM((EB,), jnp.float32),
        pltpu.VMEM((EB, H1), jnp.float32),
        pltpu.VMEM((NPAD,), jnp.float32),
        pltpu.VMEM_SHARED((NPAD, H1), jnp.float32),
        pltpu.SemaphoreType.DMA,
    ],
    compiler_params=pltpu.CompilerParams(needs_layout_passes=False),
)


def _deg_body(src_hbm, dst_hbm, dout_hbm, din_hbm,
              src_v, dst_v, dout_loc, din_loc):
    cid = lax.axis_index("c")
    sid = lax.axis_index("s")
    wid = sid * NC + cid

    zeros16 = jnp.zeros((L,), jnp.float32)

    def _z(i, carry):
        dout_loc[pl.ds(i * L, L)] = zeros16
        din_loc[pl.ds(i * L, L)] = zeros16
        return carry
    lax.fori_loop(0, NPAD // L, _z, 0)

    ones16 = jnp.ones((L,), jnp.float32)
    ebase = wid * EW

    def _batch(bi, carry):
        base = ebase + bi * EB
        pltpu.sync_copy(src_hbm.at[pl.ds(base, EB)], src_v)
        pltpu.sync_copy(dst_hbm.at[pl.ds(base, EB)], dst_v)
        for g in range(EB // L):
            sl = pl.ds(g * L, L)
            plsc.addupdate_scatter(dout_loc, [src_v[sl]], ones16)
            plsc.addupdate_scatter(din_loc, [dst_v[sl]], ones16)
        return carry
    lax.fori_loop(0, NBATCH, _batch, 0)

    pltpu.sync_copy(dout_loc, dout_hbm.at[pl.ds(wid * NPAD, NPAD)])
    pltpu.sync_copy(din_loc, din_hbm.at[pl.ds(wid * NPAD, NPAD)])


_deg_call = pl.kernel(
    _deg_body,
    out_type=(jax.ShapeDtypeStruct((NW * NPAD,), jnp.float32),
              jax.ShapeDtypeStruct((NW * NPAD,), jnp.float32)),
    mesh=_MESH,
    scratch_types=[
        pltpu.VMEM((EB,), jnp.int32),
        pltpu.VMEM((EB,), jnp.int32),
        pltpu.VMEM((NPAD,), jnp.float32),
        pltpu.VMEM((NPAD,), jnp.float32),
    ],
    compiler_params=pltpu.CompilerParams(needs_layout_passes=False),
)


def _row_spec(width=None):
    if width is None:
        return pl.BlockSpec((ROW_BLK,), lambda i: (i,))
    return pl.BlockSpec((ROW_BLK, width), lambda i: (i, 0))


def _full_spec(*shape):
    n = len(shape)
    return pl.BlockSpec(shape, lambda i: (0,) * n)


def _mm3_body(x_ref, w_ref, wal_ref, war_ref, h_ref, el_ref, er_ref):
    x = x_ref[...]
    h_ref[...] = jnp.dot(x, w_ref[...], preferred_element_type=jnp.float32)
    el_ref[...] = jnp.sum(x * wal_ref[...][None, :], axis=1)
    er_ref[...] = jnp.sum(x * war_ref[...][None, :], axis=1)


def _make_mm3(din):
    return pl.pallas_call(
        _mm3_body,
        grid=(GRID,),
        in_specs=[_row_spec(din), _full_spec(din, H1), _full_spec(din),
                  _full_spec(din)],
        out_specs=(_row_spec(H1), _row_spec(), _row_spec()),
        out_shape=(jax.ShapeDtypeStruct((NPAD, H1), jnp.float32),
                   jax.ShapeDtypeStruct((NPAD,), jnp.float32),
                   jax.ShapeDtypeStruct((NPAD,), jnp.float32)),
    )


_mm3 = _make_mm3(D)


def _gat_x(a0_ref, a1_ref, s_ref, b_ref):
    s = jnp.sum(s_ref[...], axis=0)
    s = jnp.where(s > 0.0, s, 1.0)
    agg = (a0_ref[...] + a1_ref[...]) / s[:, None]
    return jnp.maximum(agg + b_ref[...][None, :], 0.0)


def _gat_mid_body(a0_ref, a1_ref, s_ref, b_ref, w_ref, wal_ref, war_ref,
                  h_ref, el_ref, er_ref):
    x = _gat_x(a0_ref, a1_ref, s_ref, b_ref)
    h_ref[...] = jnp.dot(x, w_ref[...], preferred_element_type=jnp.float32)
    el_ref[...] = jnp.sum(x * wal_ref[...][None, :], axis=1)
    er_ref[...] = jnp.sum(x * war_ref[...][None, :], axis=1)


_gat_mid = pl.pallas_call(
    _gat_mid_body,
    grid=(GRID,),
    in_specs=[_row_spec(H1), _row_spec(H1),
              pl.BlockSpec((NW, ROW_BLK), lambda i: (0, i)),
              _full_spec(H1), _full_spec(H1, H1), _full_spec(H1),
              _full_spec(H1)],
    out_specs=(_row_spec(H1), _row_spec(), _row_spec()),
    out_shape=(jax.ShapeDtypeStruct((NPAD, H1), jnp.float32),
               jax.ShapeDtypeStruct((NPAD,), jnp.float32),
               jax.ShapeDtypeStruct((NPAD,), jnp.float32)),
)


def _gat_fc_body(a0_ref, a1_ref, s_ref, b_ref, fw_ref, fb_ref, z_ref):
    x = _gat_x(a0_ref, a1_ref, s_ref, b_ref)
    z_ref[...] = (jnp.dot(x, fw_ref[...], preferred_element_type=jnp.float32)
                  + fb_ref[...][None, :])


_gat_fc = pl.pallas_call(
    _gat_fc_body,
    grid=(GRID,),
    in_specs=[_row_spec(H1), _row_spec(H1),
              pl.BlockSpec((NW, ROW_BLK), lambda i: (0, i)),
              _full_spec(H1), _full_spec(H1, H2), _full_spec(H2)],
    out_specs=_row_spec(H2),
    out_shape=jax.ShapeDtypeStruct((NPAD, H2), jnp.float32),
)


def _sem_body(z0, z1, z2, z3, z4, p1_ref, pb1_ref, p2_ref, u_ref):
    zs = [z0[...], z1[...], z2[...], z3[...], z4[...]]
    p1 = p1_ref[...]
    pb1 = pb1_ref[...][None, :]
    p2 = p2_ref[...][None, :]
    ws = [jnp.sum(jnp.tanh(
        jnp.dot(z, p1, preferred_element_type=jnp.float32) + pb1) * p2, axis=1)
        for z in zs]
    m = ws[0]
    for w in ws[1:]:
        m = jnp.maximum(m, w)
    exs = [jnp.exp(w - m) for w in ws]
    tot = exs[0]
    for e in exs[1:]:
        tot = tot + e
    u = zs[0] * (exs[0] / tot)[:, None]
    for z, e in zip(zs[1:], exs[1:]):
        u = u + z * (e / tot)[:, None]
    u_ref[...] = u


_sem = pl.pallas_call(
    _sem_body,
    grid=(GRID,),
    in_specs=[_row_spec(H2)] * 5 + [_full_spec(H2, 128), _full_spec(128),
                                    _full_spec(128)],
    out_specs=_row_spec(H2),
    out_shape=jax.ShapeDtypeStruct((NPAD, H2), jnp.float32),
)


def _norms_body(dout_ref, din_ref, ns_ref, nd_ref):
    do = jnp.sum(dout_ref[...], axis=0)
    di = jnp.sum(din_ref[...], axis=0)
    ns_ref[...] = lax.rsqrt(jnp.maximum(do, 1.0))
    nd_ref[...] = lax.rsqrt(jnp.maximum(di, 1.0))


_norms = pl.pallas_call(
    _norms_body,
    grid=(GRID,),
    in_specs=[pl.BlockSpec((NW, ROW_BLK), lambda i: (0, i))] * 2,
    out_specs=(_row_spec(), _row_spec()),
    out_shape=(jax.ShapeDtypeStruct((NPAD,), jnp.float32),
               jax.ShapeDtypeStruct((NPAD,), jnp.float32)),
)


def _gc_in_body(u_ref, ns_ref, w_ref, h_ref):
    h_ref[...] = jnp.dot(u_ref[...] * ns_ref[...][:, None], w_ref[...],
                         preferred_element_type=jnp.float32)


_gc_in = pl.pallas_call(
    _gc_in_body,
    grid=(GRID,),
    in_specs=[_row_spec(H2), _row_spec(), _full_spec(H2, 2 * H2)],
    out_specs=_row_spec(2 * H2),
    out_shape=jax.ShapeDtypeStruct((NPAD, 2 * H2), jnp.float32),
)


def _gc_mid_body(a0_ref, a1_ref, nd_ref, ns_ref, b_ref, w_ref, h_ref):
    g = (a0_ref[...] + a1_ref[...]) * nd_ref[...][:, None] + b_ref[...][None, :]
    h_ref[...] = jnp.dot(g * ns_ref[...][:, None], w_ref[...],
                         preferred_element_type=jnp.float32)


_gc_mid = pl.pallas_call(
    _gc_mid_body,
    grid=(GRID,),
    in_specs=[_row_spec(2 * H2), _row_spec(2 * H2), _row_spec(), _row_spec(),
              _full_spec(2 * H2), _full_spec(2 * H2, 2 * H2)],
    out_specs=_row_spec(2 * H2),
    out_shape=jax.ShapeDtypeStruct((NPAD, 2 * H2), jnp.float32),
)


def _gc_fin_body(a0_ref, a1_ref, nd_ref, b_ref, noise_ref,
                 zum_ref, zuls_ref, zu_ref):
    g = (a0_ref[...] + a1_ref[...]) * nd_ref[...][:, None] + b_ref[...][None, :]
    zum = g[:, :H2]
    zuls = g[:, H2:]
    zum_ref[...] = zum
    zuls_ref[...] = zuls
    zu_ref[...] = zum + noise_ref[...] * jnp.exp(zuls)


_gc_fin = pl.pallas_call(
    _gc_fin_body,
    grid=(GRID,),
    in_specs=[_row_spec(2 * H2), _row_spec(2 * H2), _row_spec(),
              _full_spec(2 * H2), _row_spec(H2)],
    out_specs=(_row_spec(H2), _row_spec(H2), _row_spec(H2)),
    out_shape=(jax.ShapeDtypeStruct((NPAD, H2), jnp.float32),
               jax.ShapeDtypeStruct((NPAD, H2), jnp.float32),
               jax.ShapeDtypeStruct((NPAD, H2), jnp.float32)),
)


def kernel(edge_index, imagefeatures, wordfeatures, descfeatures,
           genrefeatures, compfeatures, gat_params, mean_params,
           logstd_params, sem_params):
    f32 = jnp.float32
    src = edge_index[0].astype(jnp.int32)
    dst = edge_index[1].astype(jnp.int32)
    pad = jnp.full((EPAD - E,), N, jnp.int32)
    srcp = jnp.concatenate([src, pad])
    dstp = jnp.concatenate([dst, pad])

    feats = [imagefeatures, wordfeatures, descfeatures, genrefeatures,
             compfeatures]
    zpadD = jnp.zeros((NPAD - N, D), f32)

    def agg(h, el, er):
        acc, s = _edge_call(h, el, er, srcp, dstp)
        return acc[:NPAD], acc[NPAD:], s.reshape(NW, NPAD)

    zlist = []
    for f, params in zip(feats, gat_params):
        x = jnp.concatenate([f, zpadD], axis=0)
        W1, al1, ar1, _ = params[0]
        h, el, er = _mm3(x, W1, W1 @ al1, W1 @ ar1)
        a0, a1, sg = agg(h, el, er)
        for li in range(1, 4):
            Wl, all_, arl, _ = params[li]
            b_prev = params[li - 1][3]
            h, el, er = _gat_mid(a0, a1, sg, b_prev, Wl, Wl @ all_, Wl @ arl)
            a0, a1, sg = agg(h, el, er)
        fcW, fcb = params[4]
        zlist.append(_gat_fc(a0, a1, sg, params[3][3], fcW, fcb))

    P1, pb1, P2 = sem_params
    u = _sem(*zlist, P1, pb1, P2[:, 0])

    dout, din = _deg_call(srcp, dstp)
    ns, nd = _norms(dout.reshape(NW, NPAD), din.reshape(NW, NPAD))

    Wm1, bm1, Wm2, bm2 = mean_params
    Ws1, bs1, Ws2, bs2 = logstd_params
    Wcat1 = jnp.concatenate([Wm1, Ws1], axis=1)
    bcat1 = jnp.concatenate([bm1, bs1])
    Wcat2 = jnp.zeros((2 * H2, 2 * H2), f32)
    Wcat2 = Wcat2.at[:H2, :H2].set(Wm2).at[H2:, H2:].set(Ws2)
    bcat2 = jnp.concatenate([bm2, bs2])

    zeros_n = jnp.zeros((NPAD,), f32)
    h1 = _gc_in(u, ns, Wcat1)
    g0, g1, _ = agg(h1, zeros_n, zeros_n)
    h2 = _gc_mid(g0, g1, nd, ns, bcat1, Wcat2)
    q0, q1, _ = agg(h2, zeros_n, zeros_n)

    noise = jax.random.normal(jax.random.key(42), (N, H2), dtype=f32)
    noisep = jnp.concatenate([noise, jnp.zeros((NPAD - N, H2), f32)], axis=0)
    zum, zuls, zu = _gc_fin(q0, q1, nd, bcat2, noisep)

    z = jnp.stack(zlist, axis=1)
    return (u[:N], z[:N], zu[:N], zum[:N], zuls[:N])


# speedup vs baseline: 21.3259x; 1.2971x over previous
import jax
import jax.numpy as jnp
from jax import lax
from jax.experimental import pallas as pl
from jax.experimental.pallas import tpu as pltpu
from jax.experimental.pallas import tpu_sc as plsc

N = 10000
E = 320000
D = 128
H1 = 128
H2 = 64

NC = 2
NS = 16
L = 16
NW = NC * NS

NPAD = 10240
RPT = NPAD // NS
EB = 128
TB = 160
DEG_B = 80
EPAD = 16 * TB * EB
NBB = EPAD // EB
M14 = (1 << 14) - 1

ROW_BLK = 1024
GRID = NPAD // ROW_BLK

_MESH = plsc.VectorSubcoreMesh(
    core_axis_name="c", subcore_axis_name="s", num_cores=NC, num_subcores=NS)


def _edge_body(h_hbm, el_hbm, er_hbm, pk_hbm,
               acc_hbm, s_hbm,
               el_v, er_v, pk_t, src_b, dst_b, ex_b, rows_b, s_loc, acc_sh,
               gsem, ssem):
    cid = lax.axis_index("c")
    sid = lax.axis_index("s")
    coff = cid * NPAD

    pltpu.sync_copy(el_hbm, el_v)
    pltpu.sync_copy(er_hbm, er_v)
    pltpu.sync_copy(pk_hbm.at[pl.ds(sid * TB, TB)], pk_t)

    zeros16 = jnp.zeros((L,), jnp.float32)

    def _zrow(i, carry):
        for c in range(H2 // L):
            rows_b[0][i, pl.ds(c * L, L)] = zeros16
        return carry
    lax.fori_loop(0, EB, _zrow, 0)

    def _zs(i, carry):
        s_loc[pl.ds(i * L, L)] = zeros16
        return carry
    lax.fori_loop(0, NPAD // L, _zs, 0)

    r0 = sid * RPT
    for c in range(RPT // EB):
        pltpu.sync_copy(rows_b[0], acc_sh.at[pl.ds(r0 + c * EB, EB)])
    plsc.subcore_barrier()

    def _unpack(b, k):
        for g in range(EB // L):
            sl = pl.ds(g * L, L)
            p = pk_t[b, sl]
            src_b[k][sl] = (p & M14) + coff
            dst_b[k][sl] = lax.shift_right_logical(p, 14)

    def _gather_start(k):
        pltpu.async_copy(h_hbm.at[src_b[k]], rows_b[k], gsem[k])

    def _gather_wait(k):
        pltpu.make_async_copy(h_hbm.at[src_b[k]], rows_b[k], gsem[k]).wait()

    def _scatter_drain(k):
        pltpu.make_async_copy(rows_b[k], acc_sh.at[dst_b[k]], ssem[k]).wait()

    _unpack(0, 0)
    _gather_start(0)

    def _super(i, carry):
        for k in range(2):
            b = 2 * i + k
            nk = 1 - k
            if k == 0:
                @pl.when(i > 0)
                def _():
                    _scatter_drain(nk)
                _unpack(b + 1, nk)
                _gather_start(nk)
            else:
                @pl.when(b + 1 < TB)
                def _():
                    _scatter_drain(nk)
                    _unpack(b + 1, nk)
                    _gather_start(nk)
            for g in range(EB // L):
                sl = pl.ds(g * L, L)
                p = pk_t[b, sl]
                sv = p & M14
                dv = lax.shift_right_logical(p, 14)
                e = (plsc.load_gather(el_v, [sv])
                     + plsc.load_gather(er_v, [dv]))
                e = jnp.where(e > 0.0, e, 0.1 * e)
                ex = jnp.exp(e)
                ex_b[k][sl] = ex

                @pl.when(cid == 0)
                def _():
                    plsc.addupdate_scatter(s_loc, [dv], ex)
            _gather_wait(k)

            def _scale(j4, c2):
                for r in range(4):
                    j = j4 * 4 + r
                    exb = plsc.load_gather(
                        ex_b[k], [jnp.full((L,), j, jnp.int32)])
                    for c in range(H2 // L):
                        slc = pl.ds(c * L, L)
                        rows_b[k][j, slc] = rows_b[k][j, slc] * exb
                return c2
            lax.fori_loop(0, EB // 4, _scale, 0)

            pltpu.async_copy(rows_b[k], acc_sh.at[dst_b[k]], ssem[k],
                             add=True)
        return carry
    lax.fori_loop(0, TB // 2, _super, 0)
    _scatter_drain(0)
    _scatter_drain(1)

    @pl.when(cid == 0)
    def _():
        pltpu.sync_copy(s_loc, s_hbm.at[pl.ds(sid * NPAD, NPAD)])
    plsc.subcore_barrier()

    out0 = cid * NPAD + r0
    for c in range(RPT // EB):
        pltpu.sync_copy(acc_sh.at[pl.ds(r0 + c * EB, EB)],
                        acc_hbm.at[pl.ds(out0 + c * EB, EB)])


_edge_call = pl.kernel(
    _edge_body,
    out_type=(jax.ShapeDtypeStruct((2 * NPAD, H2), jnp.float32),
              jax.ShapeDtypeStruct((NS * NPAD,), jnp.float32)),
    mesh=_MESH,
    scratch_types=[
        pltpu.VMEM((NPAD,), jnp.float32),
        pltpu.VMEM((NPAD,), jnp.float32),
        pltpu.VMEM((TB, EB), jnp.int32),
        [pltpu.VMEM((EB,), jnp.int32)] * 2,
        [pltpu.VMEM((EB,), jnp.int32)] * 2,
        [pltpu.VMEM((EB,), jnp.float32)] * 2,
        [pltpu.VMEM((EB, H2), jnp.float32)] * 2,
        pltpu.VMEM((NPAD,), jnp.float32),
        pltpu.VMEM_SHARED((NPAD, H2), jnp.float32),
        [pltpu.SemaphoreType.DMA] * 2,
        [pltpu.SemaphoreType.DMA] * 2,
    ],
    compiler_params=pltpu.CompilerParams(needs_layout_passes=False, use_tc_tiling_on_sc=False),
)


def _deg_body(pk_hbm, dout_hbm, din_hbm, pk_t, dout_loc, din_loc):
    cid = lax.axis_index("c")
    sid = lax.axis_index("s")
    wid = sid * NC + cid

    pltpu.sync_copy(pk_hbm.at[pl.ds(wid * DEG_B, DEG_B)], pk_t)

    zeros16 = jnp.zeros((L,), jnp.float32)

    def _z(i, carry):
        dout_loc[pl.ds(i * L, L)] = zeros16
        din_loc[pl.ds(i * L, L)] = zeros16
        return carry
    lax.fori_loop(0, NPAD // L, _z, 0)

    ones16 = jnp.ones((L,), jnp.float32)

    def _batch(bi, carry):
        for g in range(EB // L):
            sl = pl.ds(g * L, L)
            p = pk_t[bi, sl]
            plsc.addupdate_scatter(dout_loc, [p & M14], ones16)
            plsc.addupdate_scatter(din_loc, [lax.shift_right_logical(p, 14)],
                                   ones16)
        return carry
    lax.fori_loop(0, DEG_B, _batch, 0)

    pltpu.sync_copy(dout_loc, dout_hbm.at[pl.ds(wid * NPAD, NPAD)])
    pltpu.sync_copy(din_loc, din_hbm.at[pl.ds(wid * NPAD, NPAD)])


_deg_call = pl.kernel(
    _deg_body,
    out_type=(jax.ShapeDtypeStruct((NW * NPAD,), jnp.float32),
              jax.ShapeDtypeStruct((NW * NPAD,), jnp.float32)),
    mesh=_MESH,
    scratch_types=[
        pltpu.VMEM((DEG_B, EB), jnp.int32),
        pltpu.VMEM((NPAD,), jnp.float32),
        pltpu.VMEM((NPAD,), jnp.float32),
    ],
    compiler_params=pltpu.CompilerParams(needs_layout_passes=False, use_tc_tiling_on_sc=False),
)


def _row_spec(width=None):
    if width is None:
        return pl.BlockSpec((ROW_BLK,), lambda i: (i,))
    return pl.BlockSpec((ROW_BLK, width), lambda i: (i, 0))


def _full_spec(*shape):
    n = len(shape)
    return pl.BlockSpec(shape, lambda i: (0,) * n)


def _split_h(h):
    return jnp.stack([h[:, :H2], h[:, H2:]], axis=0)


_H_SPEC = pl.BlockSpec((2, ROW_BLK, H2), lambda i: (0, i, 0))
_H_SHAPE = jax.ShapeDtypeStruct((2, NPAD, H2), jnp.float32)
_S_SPEC = pl.BlockSpec((NS, ROW_BLK), lambda i: (0, i))


def _mm3_body(x_ref, w_ref, wal_ref, war_ref, h_ref, el_ref, er_ref):
    x = x_ref[...]
    h_ref[...] = _split_h(
        jnp.dot(x, w_ref[...], preferred_element_type=jnp.float32))
    el_ref[...] = jnp.sum(x * wal_ref[...][None, :], axis=1)
    er_ref[...] = jnp.sum(x * war_ref[...][None, :], axis=1)


_mm3 = pl.pallas_call(
    _mm3_body,
    grid=(GRID,),
    in_specs=[_row_spec(D), _full_spec(D, H1), _full_spec(D), _full_spec(D)],
    out_specs=(_H_SPEC, _row_spec(), _row_spec()),
    out_shape=(_H_SHAPE,
               jax.ShapeDtypeStruct((NPAD,), jnp.float32),
               jax.ShapeDtypeStruct((NPAD,), jnp.float32)),
)


def _gat_x(a0_ref, a1_ref, s_ref, b_ref):
    s = jnp.sum(s_ref[...], axis=0)
    s = jnp.where(s > 0.0, s, 1.0)
    agg = jnp.concatenate([a0_ref[...], a1_ref[...]], axis=1) / s[:, None]
    return jnp.maximum(agg + b_ref[...][None, :], 0.0)


def _gat_mid_body(a0_ref, a1_ref, s_ref, b_ref, w_ref, wal_ref, war_ref,
                  h_ref, el_ref, er_ref):
    x = _gat_x(a0_ref, a1_ref, s_ref, b_ref)
    h_ref[...] = _split_h(
        jnp.dot(x, w_ref[...], preferred_element_type=jnp.float32))
    el_ref[...] = jnp.sum(x * wal_ref[...][None, :], axis=1)
    er_ref[...] = jnp.sum(x * war_ref[...][None, :], axis=1)


_gat_mid = pl.pallas_call(
    _gat_mid_body,
    grid=(GRID,),
    in_specs=[_row_spec(H2), _row_spec(H2), _S_SPEC,
              _full_spec(H1), _full_spec(H1, H1), _full_spec(H1),
              _full_spec(H1)],
    out_specs=(_H_SPEC, _row_spec(), _row_spec()),
    out_shape=(_H_SHAPE,
               jax.ShapeDtypeStruct((NPAD,), jnp.float32),
               jax.ShapeDtypeStruct((NPAD,), jnp.float32)),
)


def _gat_fc_body(a0_ref, a1_ref, s_ref, b_ref, fw_ref, fb_ref, z_ref):
    x = _gat_x(a0_ref, a1_ref, s_ref, b_ref)
    z_ref[...] = (jnp.dot(x, fw_ref[...], preferred_element_type=jnp.float32)
                  + fb_ref[...][None, :])


_gat_fc = pl.pallas_call(
    _gat_fc_body,
    grid=(GRID,),
    in_specs=[_row_spec(H2), _row_spec(H2), _S_SPEC,
              _full_spec(H1), _full_spec(H1, H2), _full_spec(H2)],
    out_specs=_row_spec(H2),
    out_shape=jax.ShapeDtypeStruct((NPAD, H2), jnp.float32),
)


def _sem_body(z0, z1, z2, z3, z4, p1_ref, pb1_ref, p2_ref, u_ref):
    zs = [z0[...], z1[...], z2[...], z3[...], z4[...]]
    p1 = p1_ref[...]
    pb1 = pb1_ref[...][None, :]
    p2 = p2_ref[...][None, :]
    ws = [jnp.sum(jnp.tanh(
        jnp.dot(z, p1, preferred_element_type=jnp.float32) + pb1) * p2, axis=1)
        for z in zs]
    m = ws[0]
    for w in ws[1:]:
        m = jnp.maximum(m, w)
    exs = [jnp.exp(w - m) for w in ws]
    tot = exs[0]
    for e in exs[1:]:
        tot = tot + e
    u = zs[0] * (exs[0] / tot)[:, None]
    for z, e in zip(zs[1:], exs[1:]):
        u = u + z * (e / tot)[:, None]
    u_ref[...] = u


_sem = pl.pallas_call(
    _sem_body,
    grid=(GRID,),
    in_specs=[_row_spec(H2)] * 5 + [_full_spec(H2, 128), _full_spec(128),
                                    _full_spec(128)],
    out_specs=_row_spec(H2),
    out_shape=jax.ShapeDtypeStruct((NPAD, H2), jnp.float32),
)


def _norms_body(dout_ref, din_ref, ns_ref, nd_ref):
    do = jnp.sum(dout_ref[...], axis=0)
    di = jnp.sum(din_ref[...], axis=0)
    ns_ref[...] = lax.rsqrt(jnp.maximum(do, 1.0))
    nd_ref[...] = lax.rsqrt(jnp.maximum(di, 1.0))


_norms = pl.pallas_call(
    _norms_body,
    grid=(GRID,),
    in_specs=[pl.BlockSpec((NW, ROW_BLK), lambda i: (0, i))] * 2,
    out_specs=(_row_spec(), _row_spec()),
    out_shape=(jax.ShapeDtypeStruct((NPAD,), jnp.float32),
               jax.ShapeDtypeStruct((NPAD,), jnp.float32)),
)


def _gc_in_body(u_ref, ns_ref, w_ref, h_ref):
    h_ref[...] = _split_h(
        jnp.dot(u_ref[...] * ns_ref[...][:, None], w_ref[...],
                preferred_element_type=jnp.float32))


_gc_in = pl.pallas_call(
    _gc_in_body,
    grid=(GRID,),
    in_specs=[_row_spec(H2), _row_spec(), _full_spec(H2, 2 * H2)],
    out_specs=_H_SPEC,
    out_shape=_H_SHAPE,
)


def _gc_mid_body(a0_ref, a1_ref, nd_ref, ns_ref, b_ref, w_ref, h_ref):
    g = (jnp.concatenate([a0_ref[...], a1_ref[...]], axis=1)
         * nd_ref[...][:, None] + b_ref[...][None, :])
    h_ref[...] = _split_h(
        jnp.dot(g * ns_ref[...][:, None], w_ref[...],
                preferred_element_type=jnp.float32))


_gc_mid = pl.pallas_call(
    _gc_mid_body,
    grid=(GRID,),
    in_specs=[_row_spec(H2), _row_spec(H2), _row_spec(), _row_spec(),
              _full_spec(2 * H2), _full_spec(2 * H2, 2 * H2)],
    out_specs=_H_SPEC,
    out_shape=_H_SHAPE,
)


def _gc_fin_body(a0_ref, a1_ref, nd_ref, b0_ref, b1_ref, noise_ref,
                 zum_ref, zuls_ref, zu_ref):
    nd = nd_ref[...][:, None]
    zum = a0_ref[...] * nd + b0_ref[...][None, :]
    zuls = a1_ref[...] * nd + b1_ref[...][None, :]
    zum_ref[...] = zum
    zuls_ref[...] = zuls
    zu_ref[...] = zum + noise_ref[...] * jnp.exp(zuls)


_gc_fin = pl.pallas_call(
    _gc_fin_body,
    grid=(GRID,),
    in_specs=[_row_spec(H2), _row_spec(H2), _row_spec(),
              _full_spec(H2), _full_spec(H2), _row_spec(H2)],
    out_specs=(_row_spec(H2), _row_spec(H2), _row_spec(H2)),
    out_shape=(jax.ShapeDtypeStruct((NPAD, H2), jnp.float32),
               jax.ShapeDtypeStruct((NPAD, H2), jnp.float32),
               jax.ShapeDtypeStruct((NPAD, H2), jnp.float32)),
)


def kernel(edge_index, imagefeatures, wordfeatures, descfeatures,
           genrefeatures, compfeatures, gat_params, mean_params,
           logstd_params, sem_params):
    f32 = jnp.float32
    src = edge_index[0].astype(jnp.int32)
    dst = edge_index[1].astype(jnp.int32)
    pad = jnp.full((EPAD - E,), N, jnp.int32)
    srcp = jnp.concatenate([src, pad])
    dstp = jnp.concatenate([dst, pad])
    packed = jnp.bitwise_or(srcp, dstp << 14).reshape(NBB, EB)

    feats = [imagefeatures, wordfeatures, descfeatures, genrefeatures,
             compfeatures]
    zpadD = jnp.zeros((NPAD - N, D), f32)

    def agg(h, el, er):
        acc, s = _edge_call(h.reshape(2 * NPAD, H2), el, er, packed)
        return acc[:NPAD], acc[NPAD:], s.reshape(NS, NPAD)

    zlist = []
    for f, params in zip(feats, gat_params):
        x = jnp.concatenate([f, zpadD], axis=0)
        W1, al1, ar1, _ = params[0]
        h, el, er = _mm3(x, W1, W1 @ al1, W1 @ ar1)
        a0, a1, sg = agg(h, el, er)
        for li in range(1, 4):
            Wl, all_, arl, _ = params[li]
            b_prev = params[li - 1][3]
            h, el, er = _gat_mid(a0, a1, sg, b_prev, Wl, Wl @ all_, Wl @ arl)
            a0, a1, sg = agg(h, el, er)
        fcW, fcb = params[4]
        zlist.append(_gat_fc(a0, a1, sg, params[3][3], fcW, fcb))

    P1, pb1, P2 = sem_params
    u = _sem(*zlist, P1, pb1, P2[:, 0])

    dout, din = _deg_call(packed)
    ns, nd = _norms(dout.reshape(NW, NPAD), din.reshape(NW, NPAD))

    Wm1, bm1, Wm2, bm2 = mean_params
    Ws1, bs1, Ws2, bs2 = logstd_params
    Wcat1 = jnp.concatenate([Wm1, Ws1], axis=1)
    bcat1 = jnp.concatenate([bm1, bs1])
    Wcat2 = jnp.zeros((2 * H2, 2 * H2), f32)
    Wcat2 = Wcat2.at[:H2, :H2].set(Wm2).at[H2:, H2:].set(Ws2)
    bcat2 = jnp.concatenate([bm2, bs2])

    zeros_n = jnp.zeros((NPAD,), f32)
    h1 = _gc_in(u, ns, Wcat1)
    g0, g1, _ = agg(h1, zeros_n, zeros_n)
    h2 = _gc_mid(g0, g1, nd, ns, bcat1, Wcat2)
    q0, q1, _ = agg(h2, zeros_n, zeros_n)

    noise = jax.random.normal(jax.random.key(42), (N, H2), dtype=f32)
    noisep = jnp.concatenate([noise, jnp.zeros((NPAD - N, H2), f32)], axis=0)
    zum, zuls, zu = _gc_fin(q0, q1, nd, bm2, bs2, noisep)

    z = jnp.stack(zlist, axis=1)
    return (u[:N], z[:N], zu[:N], zum[:N], zuls[:N])


# speedup vs baseline: 23.1848x; 1.0872x over previous
import jax
import jax.numpy as jnp
from jax import lax
from jax.experimental import pallas as pl
from jax.experimental.pallas import tpu as pltpu
from jax.experimental.pallas import tpu_sc as plsc

N = 10000
E = 320000
D = 128
H1 = 128
H2 = 64

NC = 2
NS = 16
L = 16
NW = NC * NS

NPAD = 10240
RPT = NPAD // NS
EB = 128
TB = 160
DEG_B = 80
EPAD = 16 * TB * EB
NBB = EPAD // EB
M14 = (1 << 14) - 1

ROW_BLK = 1024
GRID = NPAD // ROW_BLK

_MESH = plsc.VectorSubcoreMesh(
    core_axis_name="c", subcore_axis_name="s", num_cores=NC, num_subcores=NS)


def _edge_body(h_hbm, el_hbm, er_hbm, pk_hbm,
               acc_hbm, s_hbm,
               el_v, er_v, pk_t, src_b, dst_b, ex_b, rows_b, s_loc, acc_sh,
               gsem, ssem):
    cid = lax.axis_index("c")
    sid = lax.axis_index("s")
    coff = cid * NPAD

    pltpu.sync_copy(el_hbm, el_v)
    pltpu.sync_copy(er_hbm, er_v)
    pltpu.sync_copy(pk_hbm.at[pl.ds(sid * TB, TB)], pk_t)

    zeros16 = jnp.zeros((L,), jnp.float32)

    def _zrow(i, carry):
        for c in range(H2 // L):
            rows_b[0][i, pl.ds(c * L, L)] = zeros16
        return carry
    lax.fori_loop(0, EB, _zrow, 0)

    def _zs(i, carry):
        s_loc[pl.ds(i * L, L)] = zeros16
        return carry
    lax.fori_loop(0, NPAD // L, _zs, 0)

    r0 = sid * RPT
    for c in range(RPT // EB):
        pltpu.sync_copy(rows_b[0], acc_sh.at[pl.ds(r0 + c * EB, EB)])
    plsc.subcore_barrier()

    def _unpack(b, k):
        for g in range(EB // L):
            sl = pl.ds(g * L, L)
            p = pk_t[b, sl]
            src_b[k][sl] = (p & M14) + coff
            dst_b[k][sl] = lax.shift_right_logical(p, 14)

    def _gather_start(k):
        pltpu.async_copy(h_hbm.at[src_b[k]], rows_b[k], gsem[k])

    def _gather_wait(k):
        pltpu.make_async_copy(h_hbm.at[src_b[k]], rows_b[k], gsem[k]).wait()

    def _scatter_drain(k):
        pltpu.make_async_copy(rows_b[k], acc_sh.at[dst_b[k]], ssem[k]).wait()

    NBUF = 4
    for k in range(2):
        _unpack(k, k)
        _gather_start(k)

    def _super(i, carry):
        for k in range(NBUF):
            b = NBUF * i + k
            nk = (k + 2) % NBUF
            if k < 2:
                @pl.when(i > 0)
                def _():
                    _scatter_drain(nk)
                _unpack(b + 2, nk)
                _gather_start(nk)
            else:
                _scatter_drain(nk)

                @pl.when(b + 2 < TB)
                def _():
                    _unpack(b + 2, nk)
                    _gather_start(nk)
            for g in range(EB // L):
                sl = pl.ds(g * L, L)
                p = pk_t[b, sl]
                sv = p & M14
                dv = lax.shift_right_logical(p, 14)
                e = (plsc.load_gather(el_v, [sv])
                     + plsc.load_gather(er_v, [dv]))
                e = jnp.where(e > 0.0, e, 0.1 * e)
                ex = jnp.exp(e)
                ex_b[k][sl] = ex

                @pl.when(cid == 0)
                def _():
                    plsc.addupdate_scatter(s_loc, [dv], ex)
            _gather_wait(k)

            def _scale(j4, c2):
                for r in range(4):
                    j = j4 * 4 + r
                    exb = plsc.load_gather(
                        ex_b[k], [jnp.full((L,), j, jnp.int32)])
                    for c in range(H2 // L):
                        slc = pl.ds(c * L, L)
                        rows_b[k][j, slc] = rows_b[k][j, slc] * exb
                return c2
            lax.fori_loop(0, EB // 4, _scale, 0)

            pltpu.async_copy(rows_b[k], acc_sh.at[dst_b[k]], ssem[k],
                             add=True)
        return carry
    lax.fori_loop(0, TB // NBUF, _super, 0)
    _scatter_drain(2)
    _scatter_drain(3)

    @pl.when(cid == 0)
    def _():
        pltpu.sync_copy(s_loc, s_hbm.at[pl.ds(sid * NPAD, NPAD)])
    plsc.subcore_barrier()

    out0 = cid * NPAD + r0
    for c in range(RPT // EB):
        pltpu.sync_copy(acc_sh.at[pl.ds(r0 + c * EB, EB)],
                        acc_hbm.at[pl.ds(out0 + c * EB, EB)])


_edge_call = pl.kernel(
    _edge_body,
    out_type=(jax.ShapeDtypeStruct((2 * NPAD, H2), jnp.float32),
              jax.ShapeDtypeStruct((NS * NPAD,), jnp.float32)),
    mesh=_MESH,
    scratch_types=[
        pltpu.VMEM((NPAD,), jnp.float32),
        pltpu.VMEM((NPAD,), jnp.float32),
        pltpu.VMEM((TB, EB), jnp.int32),
        [pltpu.VMEM((EB,), jnp.int32)] * 4,
        [pltpu.VMEM((EB,), jnp.int32)] * 4,
        [pltpu.VMEM((EB,), jnp.float32)] * 4,
        [pltpu.VMEM((EB, H2), jnp.float32)] * 4,
        pltpu.VMEM((NPAD,), jnp.float32),
        pltpu.VMEM_SHARED((NPAD, H2), jnp.float32),
        [pltpu.SemaphoreType.DMA] * 4,
        [pltpu.SemaphoreType.DMA] * 4,
    ],
    compiler_params=pltpu.CompilerParams(needs_layout_passes=False, use_tc_tiling_on_sc=False),
)


def _deg_body(pk_hbm, dout_hbm, din_hbm, pk_t, dout_loc, din_loc):
    cid = lax.axis_index("c")
    sid = lax.axis_index("s")
    wid = sid * NC + cid

    pltpu.sync_copy(pk_hbm.at[pl.ds(wid * DEG_B, DEG_B)], pk_t)

    zeros16 = jnp.zeros((L,), jnp.float32)

    def _z(i, carry):
        dout_loc[pl.ds(i * L, L)] = zeros16
        din_loc[pl.ds(i * L, L)] = zeros16
        return carry
    lax.fori_loop(0, NPAD // L, _z, 0)

    ones16 = jnp.ones((L,), jnp.float32)

    def _batch(bi, carry):
        for g in range(EB // L):
            sl = pl.ds(g * L, L)
            p = pk_t[bi, sl]
            plsc.addupdate_scatter(dout_loc, [p & M14], ones16)
            plsc.addupdate_scatter(din_loc, [lax.shift_right_logical(p, 14)],
                                   ones16)
        return carry
    lax.fori_loop(0, DEG_B, _batch, 0)

    pltpu.sync_copy(dout_loc, dout_hbm.at[pl.ds(wid * NPAD, NPAD)])
    pltpu.sync_copy(din_loc, din_hbm.at[pl.ds(wid * NPAD, NPAD)])


_deg_call = pl.kernel(
    _deg_body,
    out_type=(jax.ShapeDtypeStruct((NW * NPAD,), jnp.float32),
              jax.ShapeDtypeStruct((NW * NPAD,), jnp.float32)),
    mesh=_MESH,
    scratch_types=[
        pltpu.VMEM((DEG_B, EB), jnp.int32),
        pltpu.VMEM((NPAD,), jnp.float32),
        pltpu.VMEM((NPAD,), jnp.float32),
    ],
    compiler_params=pltpu.CompilerParams(needs_layout_passes=False, use_tc_tiling_on_sc=False),
)


def _row_spec(width=None):
    if width is None:
        return pl.BlockSpec((ROW_BLK,), lambda i: (i,))
    return pl.BlockSpec((ROW_BLK, width), lambda i: (i, 0))


def _full_spec(*shape):
    n = len(shape)
    return pl.BlockSpec(shape, lambda i: (0,) * n)


def _split_h(h):
    return jnp.stack([h[:, :H2], h[:, H2:]], axis=0)


_H_SPEC = pl.BlockSpec((2, ROW_BLK, H2), lambda i: (0, i, 0))
_H_SHAPE = jax.ShapeDtypeStruct((2, NPAD, H2), jnp.float32)
_S_SPEC = pl.BlockSpec((NS, ROW_BLK), lambda i: (0, i))


def _mm3_body(x_ref, w_ref, wal_ref, war_ref, h_ref, el_ref, er_ref):
    x = x_ref[...]
    h_ref[...] = _split_h(
        jnp.dot(x, w_ref[...], preferred_element_type=jnp.float32))
    el_ref[...] = jnp.sum(x * wal_ref[...][None, :], axis=1)
    er_ref[...] = jnp.sum(x * war_ref[...][None, :], axis=1)


_mm3 = pl.pallas_call(
    _mm3_body,
    grid=(GRID,),
    in_specs=[_row_spec(D), _full_spec(D, H1), _full_spec(D), _full_spec(D)],
    out_specs=(_H_SPEC, _row_spec(), _row_spec()),
    out_shape=(_H_SHAPE,
               jax.ShapeDtypeStruct((NPAD,), jnp.float32),
               jax.ShapeDtypeStruct((NPAD,), jnp.float32)),
)


def _gat_x(a0_ref, a1_ref, s_ref, b_ref):
    s = jnp.sum(s_ref[...], axis=0)
    s = jnp.where(s > 0.0, s, 1.0)
    agg = jnp.concatenate([a0_ref[...], a1_ref[...]], axis=1) / s[:, None]
    return jnp.maximum(agg + b_ref[...][None, :], 0.0)


def _gat_mid_body(a0_ref, a1_ref, s_ref, b_ref, w_ref, wal_ref, war_ref,
                  h_ref, el_ref, er_ref):
    x = _gat_x(a0_ref, a1_ref, s_ref, b_ref)
    h_ref[...] = _split_h(
        jnp.dot(x, w_ref[...], preferred_element_type=jnp.float32))
    el_ref[...] = jnp.sum(x * wal_ref[...][None, :], axis=1)
    er_ref[...] = jnp.sum(x * war_ref[...][None, :], axis=1)


_gat_mid = pl.pallas_call(
    _gat_mid_body,
    grid=(GRID,),
    in_specs=[_row_spec(H2), _row_spec(H2), _S_SPEC,
              _full_spec(H1), _full_spec(H1, H1), _full_spec(H1),
              _full_spec(H1)],
    out_specs=(_H_SPEC, _row_spec(), _row_spec()),
    out_shape=(_H_SHAPE,
               jax.ShapeDtypeStruct((NPAD,), jnp.float32),
               jax.ShapeDtypeStruct((NPAD,), jnp.float32)),
)


def _gat_fc_body(a0_ref, a1_ref, s_ref, b_ref, fw_ref, fb_ref, z_ref):
    x = _gat_x(a0_ref, a1_ref, s_ref, b_ref)
    z_ref[...] = (jnp.dot(x, fw_ref[...], preferred_element_type=jnp.float32)
                  + fb_ref[...][None, :])


_gat_fc = pl.pallas_call(
    _gat_fc_body,
    grid=(GRID,),
    in_specs=[_row_spec(H2), _row_spec(H2), _S_SPEC,
              _full_spec(H1), _full_spec(H1, H2), _full_spec(H2)],
    out_specs=_row_spec(H2),
    out_shape=jax.ShapeDtypeStruct((NPAD, H2), jnp.float32),
)


def _sem_body(z0, z1, z2, z3, z4, p1_ref, pb1_ref, p2_ref, u_ref):
    zs = [z0[...], z1[...], z2[...], z3[...], z4[...]]
    p1 = p1_ref[...]
    pb1 = pb1_ref[...][None, :]
    p2 = p2_ref[...][None, :]
    ws = [jnp.sum(jnp.tanh(
        jnp.dot(z, p1, preferred_element_type=jnp.float32) + pb1) * p2, axis=1)
        for z in zs]
    m = ws[0]
    for w in ws[1:]:
        m = jnp.maximum(m, w)
    exs = [jnp.exp(w - m) for w in ws]
    tot = exs[0]
    for e in exs[1:]:
        tot = tot + e
    u = zs[0] * (exs[0] / tot)[:, None]
    for z, e in zip(zs[1:], exs[1:]):
        u = u + z * (e / tot)[:, None]
    u_ref[...] = u


_sem = pl.pallas_call(
    _sem_body,
    grid=(GRID,),
    in_specs=[_row_spec(H2)] * 5 + [_full_spec(H2, 128), _full_spec(128),
                                    _full_spec(128)],
    out_specs=_row_spec(H2),
    out_shape=jax.ShapeDtypeStruct((NPAD, H2), jnp.float32),
)


def _norms_body(dout_ref, din_ref, ns_ref, nd_ref):
    do = jnp.sum(dout_ref[...], axis=0)
    di = jnp.sum(din_ref[...], axis=0)
    ns_ref[...] = lax.rsqrt(jnp.maximum(do, 1.0))
    nd_ref[...] = lax.rsqrt(jnp.maximum(di, 1.0))


_norms = pl.pallas_call(
    _norms_body,
    grid=(GRID,),
    in_specs=[pl.BlockSpec((NW, ROW_BLK), lambda i: (0, i))] * 2,
    out_specs=(_row_spec(), _row_spec()),
    out_shape=(jax.ShapeDtypeStruct((NPAD,), jnp.float32),
               jax.ShapeDtypeStruct((NPAD,), jnp.float32)),
)


def _gc_in_body(u_ref, ns_ref, w_ref, h_ref):
    h_ref[...] = _split_h(
        jnp.dot(u_ref[...] * ns_ref[...][:, None], w_ref[...],
                preferred_element_type=jnp.float32))


_gc_in = pl.pallas_call(
    _gc_in_body,
    grid=(GRID,),
    in_specs=[_row_spec(H2), _row_spec(), _full_spec(H2, 2 * H2)],
    out_specs=_H_SPEC,
    out_shape=_H_SHAPE,
)


def _gc_mid_body(a0_ref, a1_ref, nd_ref, ns_ref, b_ref, w_ref, h_ref):
    g = (jnp.concatenate([a0_ref[...], a1_ref[...]], axis=1)
         * nd_ref[...][:, None] + b_ref[...][None, :])
    h_ref[...] = _split_h(
        jnp.dot(g * ns_ref[...][:, None], w_ref[...],
                preferred_element_type=jnp.float32))


_gc_mid = pl.pallas_call(
    _gc_mid_body,
    grid=(GRID,),
    in_specs=[_row_spec(H2), _row_spec(H2), _row_spec(), _row_spec(),
              _full_spec(2 * H2), _full_spec(2 * H2, 2 * H2)],
    out_specs=_H_SPEC,
    out_shape=_H_SHAPE,
)


def _gc_fin_body(a0_ref, a1_ref, nd_ref, b0_ref, b1_ref, noise_ref,
                 zum_ref, zuls_ref, zu_ref):
    nd = nd_ref[...][:, None]
    zum = a0_ref[...] * nd + b0_ref[...][None, :]
    zuls = a1_ref[...] * nd + b1_ref[...][None, :]
    zum_ref[...] = zum
    zuls_ref[...] = zuls
    zu_ref[...] = zum + noise_ref[...] * jnp.exp(zuls)


_gc_fin = pl.pallas_call(
    _gc_fin_body,
    grid=(GRID,),
    in_specs=[_row_spec(H2), _row_spec(H2), _row_spec(),
              _full_spec(H2), _full_spec(H2), _row_spec(H2)],
    out_specs=(_row_spec(H2), _row_spec(H2), _row_spec(H2)),
    out_shape=(jax.ShapeDtypeStruct((NPAD, H2), jnp.float32),
               jax.ShapeDtypeStruct((NPAD, H2), jnp.float32),
               jax.ShapeDtypeStruct((NPAD, H2), jnp.float32)),
)


def kernel(edge_index, imagefeatures, wordfeatures, descfeatures,
           genrefeatures, compfeatures, gat_params, mean_params,
           logstd_params, sem_params):
    f32 = jnp.float32
    src = edge_index[0].astype(jnp.int32)
    dst = edge_index[1].astype(jnp.int32)
    pad = jnp.full((EPAD - E,), N, jnp.int32)
    srcp = jnp.concatenate([src, pad])
    dstp = jnp.concatenate([dst, pad])
    packed = jnp.bitwise_or(srcp, dstp << 14).reshape(NBB, EB)

    feats = [imagefeatures, wordfeatures, descfeatures, genrefeatures,
             compfeatures]
    zpadD = jnp.zeros((NPAD - N, D), f32)

    def agg(h, el, er):
        acc, s = _edge_call(h.reshape(2 * NPAD, H2), el, er, packed)
        return acc[:NPAD], acc[NPAD:], s.reshape(NS, NPAD)

    zlist = []
    for f, params in zip(feats, gat_params):
        x = jnp.concatenate([f, zpadD], axis=0)
        W1, al1, ar1, _ = params[0]
        h, el, er = _mm3(x, W1, W1 @ al1, W1 @ ar1)
        a0, a1, sg = agg(h, el, er)
        for li in range(1, 4):
            Wl, all_, arl, _ = params[li]
            b_prev = params[li - 1][3]
            h, el, er = _gat_mid(a0, a1, sg, b_prev, Wl, Wl @ all_, Wl @ arl)
            a0, a1, sg = agg(h, el, er)
        fcW, fcb = params[4]
        zlist.append(_gat_fc(a0, a1, sg, params[3][3], fcW, fcb))

    P1, pb1, P2 = sem_params
    u = _sem(*zlist, P1, pb1, P2[:, 0])

    dout, din = _deg_call(packed)
    ns, nd = _norms(dout.reshape(NW, NPAD), din.reshape(NW, NPAD))

    Wm1, bm1, Wm2, bm2 = mean_params
    Ws1, bs1, Ws2, bs2 = logstd_params
    Wcat1 = jnp.concatenate([Wm1, Ws1], axis=1)
    bcat1 = jnp.concatenate([bm1, bs1])
    Wcat2 = jnp.zeros((2 * H2, 2 * H2), f32)
    Wcat2 = Wcat2.at[:H2, :H2].set(Wm2).at[H2:, H2:].set(Ws2)
    bcat2 = jnp.concatenate([bm2, bs2])

    zeros_n = jnp.zeros((NPAD,), f32)
    h1 = _gc_in(u, ns, Wcat1)
    g0, g1, _ = agg(h1, zeros_n, zeros_n)
    h2 = _gc_mid(g0, g1, nd, ns, bcat1, Wcat2)
    q0, q1, _ = agg(h2, zeros_n, zeros_n)

    noise = jax.random.normal(jax.random.key(42), (N, H2), dtype=f32)
    noisep = jnp.concatenate([noise, jnp.zeros((NPAD - N, H2), f32)], axis=0)
    zum, zuls, zu = _gc_fin(q0, q1, nd, bm2, bs2, noisep)

    z = jnp.stack(zlist, axis=1)
    return (u[:N], z[:N], zu[:N], zum[:N], zuls[:N])
